# baseline (speedup 1.0000x reference)
import functools

import jax
import jax.numpy as jnp
from jax import lax
from jax.experimental import pallas as pl
from jax.experimental.pallas import tpu as pltpu
from jax.experimental.pallas import tpu_sc as plsc

_NUM_CLASSES = 100000
_D = 64
_B = 16384
_LAMBDA_C = 0.003

_INFO = plsc.get_sparse_core_info()
_NC, _NS, _L = _INFO.num_cores, _INFO.num_subcores, _INFO.num_lanes
_NW = _NC * _NS
_BPW = _B // _NW
_GCHUNK = 128
_NG = _BPW // _GCHUNK


@functools.partial(
    pl.kernel,
    out_type=jax.ShapeDtypeStruct((_B, 2 * _D), jnp.float32),
    mesh=plsc.VectorSubcoreMesh(core_axis_name="c", subcore_axis_name="s"),
    scratch_types=[
        pltpu.VMEM((_BPW,), jnp.int32),
        pltpu.VMEM((_BPW, _D), jnp.float32),
        pltpu.SemaphoreType.DMA,
    ],
    compiler_params=pltpu.CompilerParams(use_tc_tiling_on_sc=False),
)
def _gather_sc(idx_hbm, centers_hbm, out_hbm, idx_v, rows_v, gsem):
    wid = lax.axis_index("s") * _NC + lax.axis_index("c")
    base = wid * _BPW

    pltpu.sync_copy(idx_hbm.at[pl.ds(base, _BPW)], idx_v)
    gathers = [
        pltpu.async_copy(
            centers_hbm.at[idx_v.at[pl.ds(j * _GCHUNK, _GCHUNK)]],
            rows_v.at[pl.ds(j * _GCHUNK, _GCHUNK)],
            gsem,
        )
        for j in range(_NG)
    ]
    for g in gathers:
        g.wait()
    pltpu.sync_copy(rows_v, out_hbm.at[pl.ds(base, _BPW), pl.ds(0, _D)])


_TBLK = _B // 32


def _mse_tc_body(f_ref, g_ref, o_ref):
    step = pl.program_id(0)

    @pl.when(step == 0)
    def _init():
        o_ref[0, 0] = 0.0

    d = f_ref[...] - g_ref[:, : _D]
    o_ref[0, 0] += jnp.sum(d * d)

    @pl.when(step == pl.num_programs(0) - 1)
    def _scale():
        o_ref[0, 0] = o_ref[0, 0] * (_LAMBDA_C / float(_B * _D))


_mse_tc = pl.pallas_call(
    _mse_tc_body,
    grid=(_B // _TBLK,),
    in_specs=[
        pl.BlockSpec((_TBLK, _D), lambda i: (i, 0)),
        pl.BlockSpec((_TBLK, 2 * _D), lambda i: (i, 0)),
    ],
    out_specs=pl.BlockSpec(memory_space=pltpu.SMEM),
    out_shape=jax.ShapeDtypeStruct((1, 1), jnp.float32),
)


def kernel(features, labels, centers):
    gathered = _gather_sc(labels.astype(jnp.int32), centers)
    return _mse_tc(features, gathered).reshape(())


# speedup vs baseline: 2.2272x; 2.2272x over previous
import functools

import jax
import jax.numpy as jnp
from jax import lax
from jax.experimental import pallas as pl
from jax.experimental.pallas import tpu as pltpu
from jax.experimental.pallas import tpu_sc as plsc

_NUM_CLASSES = 100000
_D = 64
_B = 16384
_LAMBDA_C = 0.003

_INFO = plsc.get_sparse_core_info()
_NC, _NS, _L = _INFO.num_cores, _INFO.num_subcores, _INFO.num_lanes
_NW = _NC * _NS
_ROUNDS = _D // _NW
_BCHUNK = 8192


@functools.partial(
    pl.kernel,
    out_type=jax.ShapeDtypeStruct((_NW * _L,), jnp.float32),
    mesh=plsc.VectorSubcoreMesh(core_axis_name="c", subcore_axis_name="s"),
    scratch_types=[
        pltpu.VMEM((_BCHUNK,), jnp.int32),
        pltpu.VMEM((_NUM_CLASSES,), jnp.float32),
        pltpu.VMEM((_B,), jnp.float32),
        pltpu.VMEM((_L,), jnp.float32),
        pltpu.SemaphoreType.DMA,
    ],
    compiler_params=pltpu.CompilerParams(needs_layout_passes=False),
)
def _center_loss_sc(ft_hbm, idx_hbm, ct_hbm, out_hbm,
                    idx_v, ct_v, ft_v, part_v, sem):
    wid = lax.axis_index("s") * _NC + lax.axis_index("c")

    def round_body(r, acc):
        d = r * _NW + wid
        ct_cp = pltpu.async_copy(ct_hbm.at[d], ct_v, sem)
        ft_cp = pltpu.async_copy(ft_hbm.at[d], ft_v, sem)
        ct_cp.wait()
        ft_cp.wait()

        def chunk_body(k, acc):
            pltpu.sync_copy(idx_hbm.at[pl.ds(k * _BCHUNK, _BCHUNK)], idx_v)

            def grp_body(g, acc):
                labs = idx_v[pl.ds(g * _L, _L)]
                c = plsc.load_gather(ct_v, [labs])
                f = ft_v[pl.ds(k * _BCHUNK + g * _L, _L)]
                e = f - c
                return acc + e * e

            return lax.fori_loop(0, _BCHUNK // _L, grp_body, acc)

        return lax.fori_loop(0, _B // _BCHUNK, chunk_body, acc)

    acc = lax.fori_loop(0, _ROUNDS, round_body, jnp.zeros((_L,), jnp.float32))
    part_v[...] = acc * (_LAMBDA_C / float(_B * _D))
    pltpu.sync_copy(part_v, out_hbm.at[pl.ds(wid * _L, _L)])


def kernel(features, labels, centers):
    partials = _center_loss_sc(
        features.T, labels.astype(jnp.int32), centers.T)
    return jnp.sum(partials)


# speedup vs baseline: 2.7510x; 1.2352x over previous
import functools

import jax
import jax.numpy as jnp
from jax import lax
from jax.experimental import pallas as pl
from jax.experimental.pallas import tpu as pltpu
from jax.experimental.pallas import tpu_sc as plsc

_NUM_CLASSES = 100000
_D = 64
_B = 16384
_LAMBDA_C = 0.003

_INFO = plsc.get_sparse_core_info()
_NC, _NS, _L = _INFO.num_cores, _INFO.num_subcores, _INFO.num_lanes
_NW = _NC * _NS
_ROUNDS = _D // _NW
_FCH = 4096
_NCH = _B // _FCH
_UNROLL = 4


@functools.partial(
    pl.kernel,
    out_type=jax.ShapeDtypeStruct((_NW * _L,), jnp.float32),
    mesh=plsc.VectorSubcoreMesh(core_axis_name="c", subcore_axis_name="s"),
    scratch_types=[
        pltpu.VMEM((_B,), jnp.int32),
        pltpu.VMEM((_NUM_CLASSES,), jnp.float32),
        pltpu.VMEM((2, _FCH), jnp.float32),
        pltpu.VMEM((_L,), jnp.float32),
        pltpu.SemaphoreType.DMA,
        pltpu.SemaphoreType.DMA,
        pltpu.SemaphoreType.DMA,
        pltpu.SemaphoreType.DMA,
    ],
    compiler_params=pltpu.CompilerParams(needs_layout_passes=False),
)
def _center_loss_sc(ft_hbm, idx_hbm, ct_hbm, out_hbm,
                    idx_v, ct_v, ft_v, part_v, isem, csem, fsem0, fsem1):
    wid = lax.axis_index("s") * _NC + lax.axis_index("c")
    fsems = (fsem0, fsem1)

    idx_cp = pltpu.async_copy(idx_hbm, idx_v, isem)
    ct_cp = pltpu.async_copy(ct_hbm.at[wid], ct_v, csem)
    idx_cp.wait()

    accs = [jnp.zeros((_L,), jnp.float32) for _ in range(_UNROLL)]
    for r in range(_ROUNDS):
        d = r * _NW + wid
        ft_cps = [None] * _NCH
        ft_cps[0] = pltpu.async_copy(
            ft_hbm.at[d, pl.ds(0, _FCH)], ft_v.at[0], fsems[0])
        ct_cp.wait()
        for k in range(_NCH):
            if k + 1 < _NCH:
                ft_cps[k + 1] = pltpu.async_copy(
                    ft_hbm.at[d, pl.ds((k + 1) * _FCH, _FCH)],
                    ft_v.at[(k + 1) % 2], fsems[(k + 1) % 2])
            ft_cps[k].wait()
            buf = k % 2
            kbase = k * _FCH

            def grp_body(i, accs, buf=buf, kbase=kbase):
                out = []
                for u in range(_UNROLL):
                    off = i * (_L * _UNROLL) + u * _L
                    labs = idx_v[pl.ds(kbase + off, _L)]
                    c = plsc.load_gather(ct_v, [labs])
                    f = ft_v[buf, pl.ds(off, _L)]
                    e = f - c
                    out.append(accs[u] + e * e)
                return tuple(out)

            accs = list(lax.fori_loop(
                0, _FCH // (_L * _UNROLL), grp_body, tuple(accs)))
        if r + 1 < _ROUNDS:
            ct_cp = pltpu.async_copy(
                ct_hbm.at[(r + 1) * _NW + wid], ct_v, csem)

    total = (accs[0] + accs[1]) + (accs[2] + accs[3])
    part_v[...] = total * (_LAMBDA_C / float(_B * _D))
    pltpu.sync_copy(part_v, out_hbm.at[pl.ds(wid * _L, _L)])


def kernel(features, labels, centers):
    partials = _center_loss_sc(
        features.T, labels.astype(jnp.int32), centers.T)
    return jnp.sum(partials)
